# Initial kernel scaffold; baseline (speedup 1.0000x reference)
import jax
import jax.numpy as jnp
from jax.experimental import pallas as pl


def kernel(fine_ids, coarse_ids, fine_table, coarse_table, gate_weight):
    raise NotImplementedError("write your pallas kernel here")


# speedup vs baseline: 3.3512x; 3.3512x over previous
import functools

import jax
import jax.numpy as jnp
from jax import lax
from jax.experimental import pallas as pl
from jax.experimental.pallas import tpu as pltpu
from jax.experimental.pallas import tpu_sc as plsc

B = 16384
L = 50
D = 64
N = B * L
NC, NS, LANES = 2, 16, 16
NW = NC * NS
N_PER_W = N // NW
CHUNK = 128
N_CHUNKS = N_PER_W // CHUNK
GROUPS = CHUNK // LANES

_MESH = plsc.VectorSubcoreMesh(core_axis_name="c", subcore_axis_name="s")


@functools.partial(
    pl.kernel,
    mesh=_MESH,
    compiler_params=pltpu.CompilerParams(use_tc_tiling_on_sc=False),
    out_type=[
        jax.ShapeDtypeStruct((N, D), jnp.float32),
        jax.ShapeDtypeStruct((N,), jnp.float32),
    ],
    scratch_types=[
        pltpu.VMEM((CHUNK,), jnp.int32),
        pltpu.VMEM((CHUNK,), jnp.int32),
        pltpu.VMEM((CHUNK, D), jnp.float32),
        pltpu.VMEM((CHUNK, D), jnp.float32),
        pltpu.VMEM((CHUNK, D), jnp.float32),
        pltpu.VMEM((CHUNK,), jnp.float32),
        pltpu.VMEM((D,), jnp.float32),
        pltpu.SemaphoreType.DMA,
        pltpu.SemaphoreType.DMA,
    ],
)
def _sc_fused_lookup(fine_ids_hbm, coarse_ids_hbm, fine_tab_hbm, coarse_tab_hbm,
                     gw_hbm, fused_hbm, gate_hbm,
                     fidx_v, cidx_v, frows_v, crows_v, fused_v, gate_v, gw_v,
                     sem_f, sem_c):
    wid = lax.axis_index("s") * NC + lax.axis_index("c")
    w_base = wid * N_PER_W

    pltpu.sync_copy(gw_hbm, gw_v)
    wvec = [gw_v[pl.ds(k * LANES, LANES)] for k in range(D // LANES)]

    lane_iota = lax.iota(jnp.int32, LANES)
    perms = [(lane_iota + sh) % LANES for sh in (8, 4, 2, 1)]

    dnums = lax.GatherDimensionNumbers(
        offset_dims=(), collapsed_slice_dims=(0,), start_index_map=(0,))

    def _permute(v, p):
        return lax.gather(v, p[:, None], dimension_numbers=dnums,
                          slice_sizes=(1,),
                          mode=lax.GatherScatterMode.PROMISE_IN_BOUNDS)

    def lane_sum(v):
        for p in perms:
            v = v + _permute(v, p)
        return v

    def chunk_body(t, carry):
        base = w_base + t * CHUNK
        pltpu.sync_copy(fine_ids_hbm.at[pl.ds(base, CHUNK)], fidx_v)
        pltpu.sync_copy(coarse_ids_hbm.at[pl.ds(base, CHUNK)], cidx_v)
        cp_f = pltpu.async_copy(fine_tab_hbm.at[fidx_v], frows_v, sem_f)
        cp_c = pltpu.async_copy(coarse_tab_hbm.at[cidx_v], crows_v, sem_c)
        cp_f.wait()
        cp_c.wait()

        def group_body(g, gcarry):
            gbase = g * LANES
            gate16 = jnp.zeros((LANES,), jnp.float32)
            for i in range(LANES):
                r = gbase + i
                f0 = frows_v[r, pl.ds(0, LANES)]
                f1 = frows_v[r, pl.ds(LANES, LANES)]
                f2 = frows_v[r, pl.ds(2 * LANES, LANES)]
                f3 = frows_v[r, pl.ds(3 * LANES, LANES)]
                s = f0 * wvec[0] + f1 * wvec[1] + f2 * wvec[2] + f3 * wvec[3]
                tot = lane_sum(s)
                gsc = 1.0 / (1.0 + jnp.exp(-tot))
                c0 = crows_v[r, pl.ds(0, LANES)]
                c1 = crows_v[r, pl.ds(LANES, LANES)]
                c2 = crows_v[r, pl.ds(2 * LANES, LANES)]
                c3 = crows_v[r, pl.ds(3 * LANES, LANES)]
                fused_v[r, pl.ds(0, LANES)] = c0 + gsc * (f0 - c0)
                fused_v[r, pl.ds(LANES, LANES)] = c1 + gsc * (f1 - c1)
                fused_v[r, pl.ds(2 * LANES, LANES)] = c2 + gsc * (f2 - c2)
                fused_v[r, pl.ds(3 * LANES, LANES)] = c3 + gsc * (f3 - c3)
                gate16 = jnp.where(lane_iota == i, gsc, gate16)
            gate_v[pl.ds(gbase, LANES)] = gate16
            return gcarry

        lax.fori_loop(0, GROUPS, group_body, 0)

        pltpu.sync_copy(fused_v, fused_hbm.at[pl.ds(base, CHUNK)])
        pltpu.sync_copy(gate_v, gate_hbm.at[pl.ds(base, CHUNK)])
        return carry

    lax.fori_loop(0, N_CHUNKS, chunk_body, 0)


def kernel(fine_ids, coarse_ids, fine_table, coarse_table, gate_weight):
    fids = fine_ids.reshape(N)
    cids = coarse_ids.reshape(N)
    fused, gate = _sc_fused_lookup(fids, cids, fine_table, coarse_table,
                                   gate_weight)
    return fused.reshape(B, L, D), gate.reshape(B, L, 1)


# speedup vs baseline: 4.1494x; 1.2382x over previous
import functools

import jax
import jax.numpy as jnp
from jax import lax
from jax.experimental import pallas as pl
from jax.experimental.pallas import tpu as pltpu
from jax.experimental.pallas import tpu_sc as plsc

B = 16384
L = 50
D = 64
N = B * L
NC, NS, LANES = 2, 16, 16
NW = NC * NS
N_PER_W = N // NW
CHUNK = 128
N_CHUNKS = N_PER_W // CHUNK
GROUPS = CHUNK // LANES

_MESH = plsc.VectorSubcoreMesh(core_axis_name="c", subcore_axis_name="s")


@functools.partial(
    pl.kernel,
    mesh=_MESH,
    compiler_params=pltpu.CompilerParams(use_tc_tiling_on_sc=False),
    out_type=[
        jax.ShapeDtypeStruct((N, D), jnp.float32),
        jax.ShapeDtypeStruct((N,), jnp.float32),
    ],
    scratch_types=[
        pltpu.VMEM((N_PER_W,), jnp.int32),
        pltpu.VMEM((N_PER_W,), jnp.int32),
        [pltpu.VMEM((CHUNK, D), jnp.float32)] * 2,
        [pltpu.VMEM((CHUNK, D), jnp.float32)] * 2,
        [pltpu.VMEM((CHUNK, D), jnp.float32)] * 2,
        [pltpu.VMEM((CHUNK,), jnp.float32)] * 2,
        pltpu.VMEM((D,), jnp.float32),
        [pltpu.SemaphoreType.DMA] * 2,
        [pltpu.SemaphoreType.DMA] * 2,
    ],
)
def _sc_fused_lookup(fine_ids_hbm, coarse_ids_hbm, fine_tab_hbm, coarse_tab_hbm,
                     gw_hbm, fused_hbm, gate_hbm,
                     fids_v, cids_v, frows_v, crows_v, fused_v, gate_v, gw_v,
                     sem_g, sem_w):
    wid = lax.axis_index("s") * NC + lax.axis_index("c")
    w_base = wid * N_PER_W

    pltpu.sync_copy(gw_hbm, gw_v)
    pltpu.sync_copy(fine_ids_hbm.at[pl.ds(w_base, N_PER_W)], fids_v)
    pltpu.sync_copy(coarse_ids_hbm.at[pl.ds(w_base, N_PER_W)], cids_v)
    wvec = [gw_v[pl.ds(k * LANES, LANES)] for k in range(D // LANES)]

    lane_iota = lax.iota(jnp.int32, LANES)
    perms = [(lane_iota + sh) % LANES for sh in (8, 4, 2, 1)]
    dnums = lax.GatherDimensionNumbers(
        offset_dims=(), collapsed_slice_dims=(0,), start_index_map=(0,))

    def lane_sum(v):
        for p in perms:
            v = v + lax.gather(v, p[:, None], dimension_numbers=dnums,
                               slice_sizes=(1,),
                               mode=lax.GatherScatterMode.PROMISE_IN_BOUNDS)
        return v

    def start_gather(t, b):
        lo = t * CHUNK
        pltpu.async_copy(fine_tab_hbm.at[fids_v.at[pl.ds(lo, CHUNK)]],
                         frows_v[b], sem_g[b])
        pltpu.async_copy(coarse_tab_hbm.at[cids_v.at[pl.ds(lo, CHUNK)]],
                         crows_v[b], sem_g[b])

    def wait_gather(b):
        pltpu.make_async_copy(fine_tab_hbm.at[fids_v.at[pl.ds(0, CHUNK)]],
                              frows_v[b], sem_g[b]).wait()
        pltpu.make_async_copy(coarse_tab_hbm.at[cids_v.at[pl.ds(0, CHUNK)]],
                              crows_v[b], sem_g[b]).wait()

    def start_writeback(t, b):
        base = w_base + t * CHUNK
        pltpu.async_copy(fused_v[b], fused_hbm.at[pl.ds(base, CHUNK)],
                         sem_w[b])
        pltpu.async_copy(gate_v[b], gate_hbm.at[pl.ds(base, CHUNK)],
                         sem_w[b])

    def wait_writeback(b):
        pltpu.make_async_copy(fused_v[b], fused_hbm.at[pl.ds(0, CHUNK)],
                              sem_w[b]).wait()
        pltpu.make_async_copy(gate_v[b], gate_hbm.at[pl.ds(0, CHUNK)],
                              sem_w[b]).wait()

    def compute(b):
        def group_body(g, gcarry):
            gbase = g * LANES
            gate16 = jnp.zeros((LANES,), jnp.float32)
            for i in range(LANES):
                r = gbase + i
                f0 = frows_v[b][r, pl.ds(0, LANES)]
                f1 = frows_v[b][r, pl.ds(LANES, LANES)]
                f2 = frows_v[b][r, pl.ds(2 * LANES, LANES)]
                f3 = frows_v[b][r, pl.ds(3 * LANES, LANES)]
                s = f0 * wvec[0] + f1 * wvec[1] + f2 * wvec[2] + f3 * wvec[3]
                tot = lane_sum(s)
                gsc = 1.0 / (1.0 + jnp.exp(-tot))
                c0 = crows_v[b][r, pl.ds(0, LANES)]
                c1 = crows_v[b][r, pl.ds(LANES, LANES)]
                c2 = crows_v[b][r, pl.ds(2 * LANES, LANES)]
                c3 = crows_v[b][r, pl.ds(3 * LANES, LANES)]
                fused_v[b][r, pl.ds(0, LANES)] = c0 + gsc * (f0 - c0)
                fused_v[b][r, pl.ds(LANES, LANES)] = c1 + gsc * (f1 - c1)
                fused_v[b][r, pl.ds(2 * LANES, LANES)] = c2 + gsc * (f2 - c2)
                fused_v[b][r, pl.ds(3 * LANES, LANES)] = c3 + gsc * (f3 - c3)
                gate16 = jnp.where(lane_iota == i, gsc, gate16)
            gate_v[b][pl.ds(gbase, LANES)] = gate16
            return gcarry

        lax.fori_loop(0, GROUPS, group_body, 0)

    start_gather(0, 0)
    start_gather(1, 1)

    def pair_body(t2, carry):
        for bb in range(2):
            t = 2 * t2 + bb
            wait_gather(bb)

            @pl.when(t2 > 0)
            def _():
                wait_writeback(bb)

            compute(bb)
            start_writeback(t, bb)
            t_next = jnp.minimum(t + 2, N_CHUNKS - 1)
            start_gather(t_next, bb)
        return carry

    lax.fori_loop(0, N_CHUNKS // 2, pair_body, 0)

    wait_gather(0)
    wait_gather(1)
    wait_writeback(0)
    wait_writeback(1)


def kernel(fine_ids, coarse_ids, fine_table, coarse_table, gate_weight):
    fids = fine_ids.reshape(N)
    cids = coarse_ids.reshape(N)
    fused, gate = _sc_fused_lookup(fids, cids, fine_table, coarse_table,
                                   gate_weight)
    return fused.reshape(B, L, D), gate.reshape(B, L, 1)
